# Initial kernel scaffold; baseline (speedup 1.0000x reference)
import jax
import jax.numpy as jnp
from jax.experimental import pallas as pl


def kernel(x, Wg, w1, b1, w2, b2):
    raise NotImplementedError("write your pallas kernel here")


# speedup vs baseline: 1.3315x; 1.3315x over previous
import functools

import jax
import jax.numpy as jnp
from jax.experimental import pallas as pl
from jax.experimental.pallas import tpu as pltpu

T = 2048
D = 768
F = 3072
E = 8
K = 2
N = T * K
BM = 256
R = N + E * BM
NB = R // BM


def _ffn_body(meta_ref, xs_ref, w1_ref, b1_ref, w2_ref, b2_ref, ys_ref):
    b = pl.program_id(0)
    active = meta_ref[1, b]

    @pl.when(active > 0)
    def _():
        xb = xs_ref[...]
        h = jnp.dot(xb, w1_ref[0], preferred_element_type=jnp.float32)
        h = h + b1_ref[0]
        h = h * jax.nn.sigmoid(h)
        y = jnp.dot(h, w2_ref[0], preferred_element_type=jnp.float32)
        ys_ref[...] = y + b2_ref[0]


def _grouped_ffn(xs, w1, b1, w2, b2, meta):
    grid_spec = pltpu.PrefetchScalarGridSpec(
        num_scalar_prefetch=1,
        grid=(NB,),
        in_specs=[
            pl.BlockSpec((BM, D), lambda b, m: (b, 0)),
            pl.BlockSpec((1, D, F), lambda b, m: (m[0, b], 0, 0)),
            pl.BlockSpec((1, 1, F), lambda b, m: (m[0, b], 0, 0)),
            pl.BlockSpec((1, F, D), lambda b, m: (m[0, b], 0, 0)),
            pl.BlockSpec((1, 1, D), lambda b, m: (m[0, b], 0, 0)),
        ],
        out_specs=pl.BlockSpec((BM, D), lambda b, m: (b, 0)),
    )
    return pl.pallas_call(
        _ffn_body,
        grid_spec=grid_spec,
        out_shape=jax.ShapeDtypeStruct((R, D), jnp.float32),
    )(meta, xs, w1, b1.reshape(E, 1, F), w2, b2.reshape(E, 1, D))


def kernel(x, Wg, w1, b1, w2, b2):
    logits = x @ Wg
    probs = jax.nn.softmax(logits, axis=-1)
    topv, topi = jax.lax.top_k(probs, K)
    gates = topv / jnp.sum(topv, axis=-1, keepdims=True)

    e_flat = topi.reshape(-1).astype(jnp.int32)
    onehot = jax.nn.one_hot(e_flat, E, dtype=jnp.int32)
    counts = jnp.sum(onehot, axis=0)
    padded = ((counts + BM - 1) // BM) * BM
    pad_off = jnp.concatenate([jnp.zeros((1,), jnp.int32),
                               jnp.cumsum(padded)[:-1].astype(jnp.int32)])
    rank = jnp.take_along_axis(jnp.cumsum(onehot, axis=0) - 1,
                               e_flat[:, None], axis=1)[:, 0]
    position = pad_off[e_flat] + rank
    tok_padded = jnp.zeros((R,), jnp.int32).at[position].set(
        jnp.arange(N, dtype=jnp.int32) // K)

    blk_start = jnp.arange(NB, dtype=jnp.int32) * BM
    pad_end = pad_off + counts
    exp_id = jnp.searchsorted(pad_off, blk_start, side='right').astype(jnp.int32) - 1
    exp_id = jnp.clip(exp_id, 0, E - 1)
    blk_active = (blk_start < pad_end[exp_id]).astype(jnp.int32)
    meta = jnp.stack([exp_id, blk_active])

    xs = x[tok_padded]

    ys = _grouped_ffn(xs, w1, b1, w2, b2, meta)

    y_sel = ys[position].reshape(T, K, D)
    return jnp.sum(gates[:, :, None] * y_sel, axis=1)
